# baseline (speedup 1.0000x reference)
import functools

import jax
import jax.numpy as jnp
from jax import lax
from jax.experimental import pallas as pl
from jax.experimental.pallas import tpu as pltpu, tpu_sc as plsc

N = 10000
E = 320000
D = 128
NC, NS = 2, 16
NW = NC * NS
CH = 125
ER = E // CH
NCH = ER // NW
NPAD = 10240
SROW = NPAD // NS

BLK = 1000
NB = N // BLK

_mesh = plsc.VectorSubcoreMesh(
    core_axis_name="c", subcore_axis_name="s", num_cores=NC, num_subcores=NS
)


@functools.partial(
    pl.kernel,
    out_type=jax.ShapeDtypeStruct((NC, N), jnp.float32),
    mesh=_mesh,
    scratch_types=[
        pltpu.VMEM((NCH, CH), jnp.int32),
        pltpu.VMEM((128,), jnp.float32),
        pltpu.VMEM((N,), jnp.float32),
        pltpu.VMEM_SHARED((N,), jnp.float32),
    ],
)
def _count_k(dst_hbm, out_hbm, idx_v, ones_v, zero_v, cnt_sh):
    c = lax.axis_index("c")
    s = lax.axis_index("s")
    w = c * NS + s
    for k in range(128 // 16):
        ones_v[pl.ds(k * 16, 16)] = jnp.ones((16,), jnp.float32)

    @pl.when(s == 0)
    def _zero():
        def zb(i, carry):
            zero_v[pl.ds(i * 16, 16)] = jnp.zeros((16,), jnp.float32)
            return carry

        lax.fori_loop(0, N // 16, zb, 0)
        pltpu.sync_copy(zero_v, cnt_sh)

    plsc.subcore_barrier()
    pltpu.sync_copy(dst_hbm.at[pl.ds(w * NCH, NCH)], idx_v)

    def body(j, carry):
        pltpu.sync_copy(ones_v.at[pl.ds(0, CH)], cnt_sh.at[idx_v.at[j]], add=True)
        return carry

    lax.fori_loop(0, NCH, body, 0)
    plsc.subcore_barrier()

    @pl.when(s == 0)
    def _out():
        pltpu.sync_copy(cnt_sh, out_hbm.at[c])


DH = D // 2
ECH = ER // NS


SUB = SROW // 4


@functools.partial(
    pl.kernel,
    out_type=jax.ShapeDtypeStruct((NC, NPAD), jnp.float32),
    mesh=_mesh,
    scratch_types=[
        pltpu.VMEM((ECH, CH), jnp.int32),
        pltpu.VMEM((ECH, CH), jnp.int32),
        pltpu.VMEM((CH, DH), jnp.float32),
        pltpu.VMEM((CH, DH), jnp.float32),
        pltpu.VMEM((SUB, DH), jnp.float32),
        pltpu.VMEM((SUB, DH), jnp.float32),
        pltpu.VMEM((SROW,), jnp.float32),
        pltpu.VMEM((SROW,), jnp.float32),
        pltpu.VMEM((DH,), jnp.float32),
        pltpu.VMEM((DH,), jnp.float32),
        pltpu.VMEM_SHARED((NPAD, DH), jnp.float32),
        pltpu.SemaphoreType.DMA,
        pltpu.SemaphoreType.DMA,
    ],
    compiler_params=pltpu.CompilerParams(
        use_tc_tiling_on_sc=False, needs_layout_passes=False
    ),
)
def _agg_k(src_hbm, dst_hbm, g0_hbm, g1_hbm, dinv_hbm, b1h_hbm, w2h_hbm, out_hbm,
           src_v, dst_v, rows0_v, rows1_v, accb_v, gb_v, dinv_v, qp_v,
           b1h_v, w2h_v, acc_sh, sem0, sem1):
    c = lax.axis_index("c")
    s = lax.axis_index("s")

    def zr(i, carry):
        def zk(k, carry2):
            accb_v[i, pl.ds(k * 16, 16)] = jnp.zeros((16,), jnp.float32)
            return carry2

        lax.fori_loop(0, DH // 16, zk, 0)
        return carry

    lax.fori_loop(0, SUB, zr, 0)
    for t in range(SROW // SUB):
        pltpu.sync_copy(accb_v, acc_sh.at[pl.ds(s * SROW + t * SUB, SUB)])
    plsc.subcore_barrier()

    pltpu.sync_copy(src_hbm.at[pl.ds(s * ECH, ECH)], src_v)
    pltpu.sync_copy(dst_hbm.at[pl.ds(s * ECH, ECH)], dst_v)

    def _edge_loop(g_hbm):
        pltpu.async_copy(g_hbm.at[src_v.at[0]], rows0_v, sem0)
        pltpu.async_copy(g_hbm.at[src_v.at[1]], rows1_v, sem1)

        def body(jj, carry):
            j0 = 2 * jj
            pltpu.make_async_copy(g_hbm.at[src_v.at[j0]], rows0_v, sem0).wait()
            pltpu.sync_copy(rows0_v, acc_sh.at[dst_v.at[j0]], add=True)

            @pl.when(jj < ECH // 2 - 1)
            def _n0():
                pltpu.async_copy(g_hbm.at[src_v.at[j0 + 2]], rows0_v, sem0)

            pltpu.make_async_copy(g_hbm.at[src_v.at[j0 + 1]], rows1_v, sem1).wait()
            pltpu.sync_copy(rows1_v, acc_sh.at[dst_v.at[j0 + 1]], add=True)

            @pl.when(jj < ECH // 2 - 1)
            def _n1():
                pltpu.async_copy(g_hbm.at[src_v.at[j0 + 3]], rows1_v, sem1)

            return carry

        lax.fori_loop(0, ECH // 2, body, 0)

    @pl.when(c == 0)
    def _half0():
        _edge_loop(g0_hbm)

    @pl.when(c == 1)
    def _half1():
        _edge_loop(g1_hbm)

    plsc.subcore_barrier()

    base = s * SROW
    pltpu.sync_copy(b1h_hbm.at[c], b1h_v)
    pltpu.sync_copy(w2h_hbm.at[c], w2h_v)
    pltpu.sync_copy(dinv_hbm.at[pl.ds(base, SROW)], dinv_v)

    def _epilogue(g_hbm):
        lanes = lax.iota(jnp.int32, 16)
        b1s = [b1h_v[pl.ds(j * 16, 16)] for j in range(DH // 16)]
        w2s = [w2h_v[pl.ds(j * 16, 16)] for j in range(DH // 16)]
        for t in range(SROW // SUB):
            off = base + t * SUB
            pltpu.sync_copy(acc_sh.at[pl.ds(off, SUB)], accb_v)
            pltpu.sync_copy(g_hbm.at[pl.ds(off, SUB)], gb_v)

            def gbody(rb, carry):
                r0 = rb * 16
                dvec = dinv_v[pl.ds(t * SUB + r0, 16)]
                ridx = lanes + r0
                qacc = jnp.zeros((16,), jnp.float32)
                for k in range(DH):
                    cidx = jnp.full((16,), k, jnp.int32)
                    av = plsc.load_gather(accb_v, [ridx, cidx])
                    gv = plsc.load_gather(gb_v, [ridx, cidx])
                    z = (av + gv) * dvec + b1s[k // 16][k % 16]
                    qacc = qacc + jnp.maximum(z, 0.0) * w2s[k // 16][k % 16]
                qp_v[pl.ds(t * SUB + r0, 16)] = qacc * dvec
                return carry

            lax.fori_loop(0, SUB // 16, gbody, 0)

    @pl.when(c == 0)
    def _ep0():
        _epilogue(g0_hbm)

    @pl.when(c == 1)
    def _ep1():
        _epilogue(g1_hbm)

    pltpu.sync_copy(qp_v, out_hbm.at[c, pl.ds(base, SROW)])


CH2 = 80
EPT = E // NW
NC2 = EPT // CH2


@functools.partial(
    pl.kernel,
    out_type=jax.ShapeDtypeStruct((NC, N), jnp.float32),
    mesh=_mesh,
    scratch_types=[
        pltpu.VMEM((EPT,), jnp.int32),
        pltpu.VMEM((NC2, CH2), jnp.int32),
        pltpu.VMEM((EPT,), jnp.float32),
        pltpu.VMEM((N,), jnp.float32),
        pltpu.VMEM((N,), jnp.float32),
        pltpu.VMEM_SHARED((N,), jnp.float32),
    ],
    compiler_params=pltpu.CompilerParams(needs_layout_passes=False),
)
def _sagg_k(src_hbm, dst_hbm, q_hbm, out_hbm, src_v, dst_v, vals_v, q_v, zero_v, acc_sh):
    c = lax.axis_index("c")
    s = lax.axis_index("s")
    w = c * NS + s

    @pl.when(s == 0)
    def _zero():
        def zb(i, carry):
            zero_v[pl.ds(i * 16, 16)] = jnp.zeros((16,), jnp.float32)
            return carry

        lax.fori_loop(0, N // 16, zb, 0)
        pltpu.sync_copy(zero_v, acc_sh)

    plsc.subcore_barrier()
    pltpu.sync_copy(q_hbm.at[0], q_v)
    pltpu.sync_copy(q_hbm.at[1], zero_v)

    def qadd(i, carry):
        sl = pl.ds(i * 16, 16)
        q_v[sl] = q_v[sl] + zero_v[sl]
        return carry

    lax.fori_loop(0, N // 16, qadd, 0)
    pltpu.sync_copy(src_hbm.at[pl.ds(w * EPT, EPT)], src_v)
    pltpu.sync_copy(dst_hbm.at[w], dst_v)

    def gbody(i, carry):
        iv = src_v[pl.ds(i * 16, 16)]
        vals_v[pl.ds(i * 16, 16)] = plsc.load_gather(q_v, [iv])
        return carry

    lax.fori_loop(0, EPT // 16, gbody, 0)

    def sbody(j, carry):
        pltpu.sync_copy(vals_v.at[pl.ds(j * CH2, CH2)], acc_sh.at[dst_v.at[j]], add=True)
        return carry

    lax.fori_loop(0, NC2, sbody, 0)
    plsc.subcore_barrier()

    @pl.when(s == 0)
    def _out():
        pltpu.sync_copy(acc_sh, out_hbm.at[c])


def _mm_body(x_ref, w1_ref, h_ref):
    h_ref[...] = jnp.dot(x_ref[...], w1_ref[...], preferred_element_type=jnp.float32)


def _scale_body(h_ref, cnt_ref, g0_ref, g1_ref, dinv_ref):
    deg = cnt_ref[0, 0, 0, :] + cnt_ref[1, 0, 0, :] + 1.0
    dinv = lax.rsqrt(deg)
    g = h_ref[...] * dinv[:, None]
    g0_ref[...] = g[:, :DH]
    g1_ref[...] = g[:, DH:]
    dinv_ref[0, 0, :] = dinv


def _final_body(s_ref, q_ref, dinv_ref, b2_ref, out_ref):
    tot = s_ref[0, 0, 0, :] + s_ref[1, 0, 0, :] + q_ref[0, 0, 0, :] + q_ref[1, 0, 0, :]
    out_ref[0, 0, :] = jnp.tanh(dinv_ref[0, 0, :] * tot + b2_ref[0, 0])


_mm_call = pl.pallas_call(
    _mm_body,
    grid=(NB,),
    in_specs=[
        pl.BlockSpec((BLK, D), lambda i: (i, 0)),
        pl.BlockSpec((D, D), lambda i: (0, 0)),
    ],
    out_specs=pl.BlockSpec((BLK, D), lambda i: (i, 0)),
    out_shape=jax.ShapeDtypeStruct((N, D), jnp.float32),
)

_scale_call = pl.pallas_call(
    _scale_body,
    grid=(NB,),
    in_specs=[
        pl.BlockSpec((BLK, D), lambda i: (i, 0)),
        pl.BlockSpec((NC, 1, 1, BLK), lambda i: (0, i, 0, 0)),
    ],
    out_specs=[
        pl.BlockSpec((BLK, DH), lambda i: (i, 0)),
        pl.BlockSpec((BLK, DH), lambda i: (i, 0)),
        pl.BlockSpec((1, 1, BLK), lambda i: (i, 0, 0)),
    ],
    out_shape=[
        jax.ShapeDtypeStruct((NPAD, DH), jnp.float32),
        jax.ShapeDtypeStruct((NPAD, DH), jnp.float32),
        jax.ShapeDtypeStruct((NB, 1, BLK), jnp.float32),
    ],
)

_final_call = pl.pallas_call(
    _final_body,
    grid=(NB,),
    in_specs=[
        pl.BlockSpec((NC, 1, 1, BLK), lambda i: (0, i, 0, 0)),
        pl.BlockSpec((NC, 1, 1, BLK), lambda i: (0, i, 0, 0)),
        pl.BlockSpec((1, 1, BLK), lambda i: (i, 0, 0)),
        pl.BlockSpec((1, 1), lambda i: (0, 0)),
    ],
    out_specs=pl.BlockSpec((1, 1, BLK), lambda i: (i, 0, 0)),
    out_shape=jax.ShapeDtypeStruct((NB, 1, BLK), jnp.float32),
)


def kernel(x, edge_index, W1, b1, W2, b2):
    ei = edge_index.astype(jnp.int32)
    src2 = ei[0].reshape(ER, CH)
    dst2 = ei[1].reshape(ER, CH)

    counts = _count_k(dst2)
    h = _mm_call(x, W1)
    g0, g1, dinv3 = _scale_call(h, counts.reshape(NC, NB, 1, BLK))
    dinvp = jnp.pad(dinv3.reshape(N), (0, NPAD - N))
    qp = _agg_k(src2, dst2, g0, g1, dinvp,
                b1.reshape(NC, DH), W2.reshape(NC, DH))
    qs = qp[:, :N]
    s_part = _sagg_k(ei[0], ei[1].reshape(NW, NC2, CH2), qs)
    out3 = _final_call(
        s_part.reshape(NC, NB, 1, BLK),
        qs.reshape(NC, NB, 1, BLK),
        dinv3,
        b2.reshape(1, 1),
    )
    return out3.reshape(N, 1)


# speedup vs baseline: 1.1371x; 1.1371x over previous
import functools

import jax
import jax.numpy as jnp
from jax import lax
from jax.experimental import pallas as pl
from jax.experimental.pallas import tpu as pltpu, tpu_sc as plsc

N = 10000
E = 320000
D = 128
NC, NS = 2, 16
NW = NC * NS
CH = 125
ER = E // CH
NCH = ER // NW
NPAD = 10240
SROW = NPAD // NS

BLK = 1000
NB = N // BLK

_mesh = plsc.VectorSubcoreMesh(
    core_axis_name="c", subcore_axis_name="s", num_cores=NC, num_subcores=NS
)


@functools.partial(
    pl.kernel,
    out_type=jax.ShapeDtypeStruct((NC, N), jnp.float32),
    mesh=_mesh,
    scratch_types=[
        pltpu.VMEM((NCH, CH), jnp.int32),
        pltpu.VMEM((128,), jnp.float32),
        pltpu.VMEM((N,), jnp.float32),
        pltpu.VMEM_SHARED((N,), jnp.float32),
    ],
)
def _count_k(dst_hbm, out_hbm, idx_v, ones_v, zero_v, cnt_sh):
    c = lax.axis_index("c")
    s = lax.axis_index("s")
    w = c * NS + s
    for k in range(128 // 16):
        ones_v[pl.ds(k * 16, 16)] = jnp.ones((16,), jnp.float32)

    @pl.when(s == 0)
    def _zero():
        def zb(i, carry):
            zero_v[pl.ds(i * 16, 16)] = jnp.zeros((16,), jnp.float32)
            return carry

        lax.fori_loop(0, N // 16, zb, 0)
        pltpu.sync_copy(zero_v, cnt_sh)

    plsc.subcore_barrier()
    pltpu.sync_copy(dst_hbm.at[pl.ds(w * NCH, NCH)], idx_v)

    def body(j, carry):
        pltpu.sync_copy(ones_v.at[pl.ds(0, CH)], cnt_sh.at[idx_v.at[j]], add=True)
        return carry

    lax.fori_loop(0, NCH, body, 0)
    plsc.subcore_barrier()

    @pl.when(s == 0)
    def _out():
        pltpu.sync_copy(cnt_sh, out_hbm.at[c])


DH = D // 2
ECH = ER // NS


SUB = SROW // 8


@functools.partial(
    pl.kernel,
    out_type=jax.ShapeDtypeStruct((NC, NPAD), jnp.float32),
    mesh=_mesh,
    scratch_types=[
        pltpu.VMEM((ECH, CH), jnp.int32),
        pltpu.VMEM((ECH, CH), jnp.int32),
        pltpu.VMEM((CH, DH), jnp.float32),
        pltpu.VMEM((CH, DH), jnp.float32),
        pltpu.VMEM((SUB, DH), jnp.float32),
        pltpu.VMEM((SUB, DH), jnp.float32),
        pltpu.VMEM((SROW, 16), jnp.float32),
        pltpu.VMEM((SROW,), jnp.float32),
        pltpu.VMEM((DH,), jnp.float32),
        pltpu.VMEM((DH,), jnp.float32),
        pltpu.VMEM_SHARED((NPAD, DH), jnp.float32),
        pltpu.SemaphoreType.DMA,
        pltpu.SemaphoreType.DMA,
    ],
    compiler_params=pltpu.CompilerParams(
        use_tc_tiling_on_sc=False, needs_layout_passes=False
    ),
)
def _agg_k(src_hbm, dst_hbm, g0_hbm, g1_hbm, dm_hbm, b1h_hbm, w2h_hbm, out_hbm,
           src_v, dst_v, rows0_v, rows1_v, accb_v, gb_v, dm_v, qp_v,
           b1h_v, w2h_v, acc_sh, sem0, sem1):
    c = lax.axis_index("c")
    s = lax.axis_index("s")

    def zr(i, carry):
        def zk(k, carry2):
            accb_v[i, pl.ds(k * 16, 16)] = jnp.zeros((16,), jnp.float32)
            return carry2

        lax.fori_loop(0, DH // 16, zk, 0)
        return carry

    lax.fori_loop(0, SUB, zr, 0)
    for t in range(SROW // SUB):
        pltpu.sync_copy(accb_v, acc_sh.at[pl.ds(s * SROW + t * SUB, SUB)])
    plsc.subcore_barrier()

    pltpu.sync_copy(src_hbm.at[pl.ds(s * ECH, ECH)], src_v)
    pltpu.sync_copy(dst_hbm.at[pl.ds(s * ECH, ECH)], dst_v)

    def _edge_loop(g_hbm):
        pltpu.async_copy(g_hbm.at[src_v.at[0]], rows0_v, sem0)
        pltpu.async_copy(g_hbm.at[src_v.at[1]], rows1_v, sem1)

        def body(jj, carry):
            j0 = 2 * jj
            pltpu.make_async_copy(g_hbm.at[src_v.at[j0]], rows0_v, sem0).wait()
            pltpu.sync_copy(rows0_v, acc_sh.at[dst_v.at[j0]], add=True)

            @pl.when(jj < ECH // 2 - 1)
            def _n0():
                pltpu.async_copy(g_hbm.at[src_v.at[j0 + 2]], rows0_v, sem0)

            pltpu.make_async_copy(g_hbm.at[src_v.at[j0 + 1]], rows1_v, sem1).wait()
            pltpu.sync_copy(rows1_v, acc_sh.at[dst_v.at[j0 + 1]], add=True)

            @pl.when(jj < ECH // 2 - 1)
            def _n1():
                pltpu.async_copy(g_hbm.at[src_v.at[j0 + 3]], rows1_v, sem1)

            return carry

        lax.fori_loop(0, ECH // 2, body, 0)

    @pl.when(c == 0)
    def _half0():
        _edge_loop(g0_hbm)

    @pl.when(c == 1)
    def _half1():
        _edge_loop(g1_hbm)

    plsc.subcore_barrier()

    base = s * SROW
    pltpu.sync_copy(b1h_hbm.at[c], b1h_v)
    pltpu.sync_copy(w2h_hbm.at[c], w2h_v)
    pltpu.sync_copy(dm_hbm.at[pl.ds(base, SROW)], dm_v)

    def _epilogue(g_hbm):
        lanes = lax.iota(jnp.int32, 16)
        b1s = [b1h_v[pl.ds(j * 16, 16)] for j in range(DH // 16)]
        w2s = [w2h_v[pl.ds(j * 16, 16)] for j in range(DH // 16)]
        for t in range(SROW // SUB):
            off = base + t * SUB
            pltpu.sync_copy(acc_sh.at[pl.ds(off, SUB)], accb_v)
            pltpu.sync_copy(g_hbm.at[pl.ds(off, SUB)], gb_v)

            def gbody(rb, carry):
                r0 = rb * 16

                def rbody(i, qvec):
                    r = r0 + i
                    dvr = dm_v[t * SUB + r, pl.ds(0, 16)]
                    y = jnp.zeros((16,), jnp.float32)
                    for j in range(DH // 16):
                        sl = pl.ds(j * 16, 16)
                        z = (accb_v[r, sl] + gb_v[r, sl]) * dvr + b1s[j]
                        y = y + jnp.maximum(z, 0.0) * w2s[j]
                    qr = jnp.sum(y * dvr)
                    return jnp.where(lanes == i, qr, qvec)

                qvec = lax.fori_loop(0, 16, rbody, jnp.zeros((16,), jnp.float32))
                qp_v[pl.ds(t * SUB + r0, 16)] = qvec
                return carry

            lax.fori_loop(0, SUB // 16, gbody, 0)

    @pl.when(c == 0)
    def _ep0():
        _epilogue(g0_hbm)

    @pl.when(c == 1)
    def _ep1():
        _epilogue(g1_hbm)

    pltpu.sync_copy(qp_v, out_hbm.at[c, pl.ds(base, SROW)])


CH2 = 80
EPT = E // NW
NC2 = EPT // CH2


@functools.partial(
    pl.kernel,
    out_type=jax.ShapeDtypeStruct((NC, N), jnp.float32),
    mesh=_mesh,
    scratch_types=[
        pltpu.VMEM((EPT,), jnp.int32),
        pltpu.VMEM((NC2, CH2), jnp.int32),
        pltpu.VMEM((EPT,), jnp.float32),
        pltpu.VMEM((N,), jnp.float32),
        pltpu.VMEM((N,), jnp.float32),
        pltpu.VMEM_SHARED((N,), jnp.float32),
    ],
    compiler_params=pltpu.CompilerParams(needs_layout_passes=False),
)
def _sagg_k(src_hbm, dst_hbm, q_hbm, out_hbm, src_v, dst_v, vals_v, q_v, zero_v, acc_sh):
    c = lax.axis_index("c")
    s = lax.axis_index("s")
    w = c * NS + s

    @pl.when(s == 0)
    def _zero():
        def zb(i, carry):
            zero_v[pl.ds(i * 16, 16)] = jnp.zeros((16,), jnp.float32)
            return carry

        lax.fori_loop(0, N // 16, zb, 0)
        pltpu.sync_copy(zero_v, acc_sh)

    plsc.subcore_barrier()
    pltpu.sync_copy(q_hbm.at[0], q_v)
    pltpu.sync_copy(q_hbm.at[1], zero_v)

    def qadd(i, carry):
        sl = pl.ds(i * 16, 16)
        q_v[sl] = q_v[sl] + zero_v[sl]
        return carry

    lax.fori_loop(0, N // 16, qadd, 0)
    pltpu.sync_copy(src_hbm.at[pl.ds(w * EPT, EPT)], src_v)
    pltpu.sync_copy(dst_hbm.at[w], dst_v)

    def gbody(i, carry):
        iv = src_v[pl.ds(i * 16, 16)]
        vals_v[pl.ds(i * 16, 16)] = plsc.load_gather(q_v, [iv])
        return carry

    lax.fori_loop(0, EPT // 16, gbody, 0)

    def sbody(j, carry):
        pltpu.sync_copy(vals_v.at[pl.ds(j * CH2, CH2)], acc_sh.at[dst_v.at[j]], add=True)
        return carry

    lax.fori_loop(0, NC2, sbody, 0)
    plsc.subcore_barrier()

    @pl.when(s == 0)
    def _out():
        pltpu.sync_copy(acc_sh, out_hbm.at[c])


def _mm_body(x_ref, w1_ref, h_ref):
    h_ref[...] = jnp.dot(x_ref[...], w1_ref[...], preferred_element_type=jnp.float32)


def _scale_body(h_ref, cnt_ref, g0_ref, g1_ref, dinv_ref, dm_ref):
    deg = cnt_ref[0, 0, 0, :] + cnt_ref[1, 0, 0, :] + 1.0
    dinv = lax.rsqrt(deg)
    g = h_ref[...] * dinv[:, None]
    g0_ref[...] = g[:, :DH]
    g1_ref[...] = g[:, DH:]
    dinv_ref[0, 0, :] = dinv
    dm_ref[...] = jnp.broadcast_to(dinv[:, None], (BLK, 16))


def _final_body(s_ref, q_ref, dinv_ref, b2_ref, out_ref):
    tot = s_ref[0, 0, 0, :] + s_ref[1, 0, 0, :] + q_ref[0, 0, 0, :] + q_ref[1, 0, 0, :]
    out_ref[0, 0, :] = jnp.tanh(dinv_ref[0, 0, :] * tot + b2_ref[0, 0])


_mm_call = pl.pallas_call(
    _mm_body,
    grid=(NB,),
    in_specs=[
        pl.BlockSpec((BLK, D), lambda i: (i, 0)),
        pl.BlockSpec((D, D), lambda i: (0, 0)),
    ],
    out_specs=pl.BlockSpec((BLK, D), lambda i: (i, 0)),
    out_shape=jax.ShapeDtypeStruct((N, D), jnp.float32),
)

_scale_call = pl.pallas_call(
    _scale_body,
    grid=(NB,),
    in_specs=[
        pl.BlockSpec((BLK, D), lambda i: (i, 0)),
        pl.BlockSpec((NC, 1, 1, BLK), lambda i: (0, i, 0, 0)),
    ],
    out_specs=[
        pl.BlockSpec((BLK, DH), lambda i: (i, 0)),
        pl.BlockSpec((BLK, DH), lambda i: (i, 0)),
        pl.BlockSpec((1, 1, BLK), lambda i: (i, 0, 0)),
        pl.BlockSpec((BLK, 16), lambda i: (i, 0)),
    ],
    out_shape=[
        jax.ShapeDtypeStruct((NPAD, DH), jnp.float32),
        jax.ShapeDtypeStruct((NPAD, DH), jnp.float32),
        jax.ShapeDtypeStruct((NB, 1, BLK), jnp.float32),
        jax.ShapeDtypeStruct((NPAD, 16), jnp.float32),
    ],
)

_final_call = pl.pallas_call(
    _final_body,
    grid=(NB,),
    in_specs=[
        pl.BlockSpec((NC, 1, 1, BLK), lambda i: (0, i, 0, 0)),
        pl.BlockSpec((NC, 1, 1, BLK), lambda i: (0, i, 0, 0)),
        pl.BlockSpec((1, 1, BLK), lambda i: (i, 0, 0)),
        pl.BlockSpec((1, 1), lambda i: (0, 0)),
    ],
    out_specs=pl.BlockSpec((1, 1, BLK), lambda i: (i, 0, 0)),
    out_shape=jax.ShapeDtypeStruct((NB, 1, BLK), jnp.float32),
)


def kernel(x, edge_index, W1, b1, W2, b2):
    ei = edge_index.astype(jnp.int32)
    src2 = ei[0].reshape(ER, CH)
    dst2 = ei[1].reshape(ER, CH)

    counts = _count_k(dst2)
    h = _mm_call(x, W1)
    g0, g1, dinv3, dm = _scale_call(h, counts.reshape(NC, NB, 1, BLK))
    qp = _agg_k(src2, dst2, g0, g1, dm,
                b1.reshape(NC, DH), W2.reshape(NC, DH))
    qs = qp[:, :N]
    s_part = _sagg_k(ei[0], ei[1].reshape(NW, NC2, CH2), qs)
    out3 = _final_call(
        s_part.reshape(NC, NB, 1, BLK),
        qs.reshape(NC, NB, 1, BLK),
        dinv3,
        b2.reshape(1, 1),
    )
    return out3.reshape(N, 1)


# speedup vs baseline: 1.1950x; 1.0509x over previous
import functools

import jax
import jax.numpy as jnp
from jax import lax
from jax.experimental import pallas as pl
from jax.experimental.pallas import tpu as pltpu, tpu_sc as plsc

N = 10000
E = 320000
D = 128
NC, NS = 2, 16
NW = NC * NS
CH = 125
ER = E // CH
NCH = ER // NW
NPAD = 10240
SROW = NPAD // NS

BLK = 1000
NB = N // BLK

_mesh = plsc.VectorSubcoreMesh(
    core_axis_name="c", subcore_axis_name="s", num_cores=NC, num_subcores=NS
)


@functools.partial(
    pl.kernel,
    out_type=jax.ShapeDtypeStruct((NC, N), jnp.float32),
    mesh=_mesh,
    scratch_types=[
        pltpu.VMEM((NCH, CH), jnp.int32),
        pltpu.VMEM((128,), jnp.float32),
        pltpu.VMEM((N,), jnp.float32),
        pltpu.VMEM_SHARED((N,), jnp.float32),
    ],
    compiler_params=pltpu.CompilerParams(use_tc_tiling_on_sc=False),
)
def _count_k(edge_hbm, out_hbm, idx_v, ones_v, zero_v, cnt_sh):
    c = lax.axis_index("c")
    s = lax.axis_index("s")
    w = c * NS + s
    for k in range(128 // 16):
        ones_v[pl.ds(k * 16, 16)] = jnp.ones((16,), jnp.float32)

    @pl.when(s == 0)
    def _zero():
        def zb(i, carry):
            zero_v[pl.ds(i * 16, 16)] = jnp.zeros((16,), jnp.float32)
            return carry

        lax.fori_loop(0, N // 16, zb, 0)
        pltpu.sync_copy(zero_v, cnt_sh)

    plsc.subcore_barrier()
    pltpu.sync_copy(edge_hbm.at[1, pl.ds(w * NCH, NCH)], idx_v)

    def body(j, carry):
        pltpu.sync_copy(ones_v.at[pl.ds(0, CH)], cnt_sh.at[idx_v.at[j]], add=True)
        return carry

    lax.fori_loop(0, NCH, body, 0)
    plsc.subcore_barrier()

    @pl.when(s == 0)
    def _out():
        pltpu.sync_copy(cnt_sh, out_hbm.at[c])


DH = D // 2
ECH = ER // NS


SUB = SROW // 8


@functools.partial(
    pl.kernel,
    out_type=jax.ShapeDtypeStruct((NC, NPAD), jnp.float32),
    mesh=_mesh,
    scratch_types=[
        pltpu.VMEM((ECH, CH), jnp.int32),
        pltpu.VMEM((ECH, CH), jnp.int32),
        pltpu.VMEM((CH, DH), jnp.float32),
        pltpu.VMEM((CH, DH), jnp.float32),
        pltpu.VMEM((SUB, DH), jnp.float32),
        pltpu.VMEM((SUB, DH), jnp.float32),
        pltpu.VMEM((SROW, 16), jnp.float32),
        pltpu.VMEM((SROW,), jnp.float32),
        pltpu.VMEM((DH,), jnp.float32),
        pltpu.VMEM((DH,), jnp.float32),
        pltpu.VMEM_SHARED((NPAD, DH), jnp.float32),
        pltpu.SemaphoreType.DMA,
        pltpu.SemaphoreType.DMA,
    ],
    compiler_params=pltpu.CompilerParams(
        use_tc_tiling_on_sc=False, needs_layout_passes=False
    ),
)
def _agg_k(edge_hbm, g0_hbm, g1_hbm, dm_hbm, b1h_hbm, w2h_hbm, out_hbm,
           src_v, dst_v, rows0_v, rows1_v, accb_v, gb_v, dm_v, qp_v,
           b1h_v, w2h_v, acc_sh, sem0, sem1):
    c = lax.axis_index("c")
    s = lax.axis_index("s")

    def zr(i, carry):
        def zk(k, carry2):
            accb_v[i, pl.ds(k * 16, 16)] = jnp.zeros((16,), jnp.float32)
            return carry2

        lax.fori_loop(0, DH // 16, zk, 0)
        return carry

    lax.fori_loop(0, SUB, zr, 0)
    for t in range(SROW // SUB):
        pltpu.sync_copy(accb_v, acc_sh.at[pl.ds(s * SROW + t * SUB, SUB)])
    plsc.subcore_barrier()

    pltpu.sync_copy(edge_hbm.at[0, pl.ds(s * ECH, ECH)], src_v)
    pltpu.sync_copy(edge_hbm.at[1, pl.ds(s * ECH, ECH)], dst_v)

    def _edge_loop(g_hbm):
        pltpu.async_copy(g_hbm.at[src_v.at[0]], rows0_v, sem0)
        pltpu.async_copy(g_hbm.at[src_v.at[1]], rows1_v, sem1)

        def body(jj, carry):
            j0 = 2 * jj
            pltpu.make_async_copy(g_hbm.at[src_v.at[j0]], rows0_v, sem0).wait()
            pltpu.sync_copy(rows0_v, acc_sh.at[dst_v.at[j0]], add=True)

            @pl.when(jj < ECH // 2 - 1)
            def _n0():
                pltpu.async_copy(g_hbm.at[src_v.at[j0 + 2]], rows0_v, sem0)

            pltpu.make_async_copy(g_hbm.at[src_v.at[j0 + 1]], rows1_v, sem1).wait()
            pltpu.sync_copy(rows1_v, acc_sh.at[dst_v.at[j0 + 1]], add=True)

            @pl.when(jj < ECH // 2 - 1)
            def _n1():
                pltpu.async_copy(g_hbm.at[src_v.at[j0 + 3]], rows1_v, sem1)

            return carry

        lax.fori_loop(0, ECH // 2, body, 0)

    @pl.when(c == 0)
    def _half0():
        _edge_loop(g0_hbm)

    @pl.when(c == 1)
    def _half1():
        _edge_loop(g1_hbm)

    plsc.subcore_barrier()

    base = s * SROW
    pltpu.sync_copy(b1h_hbm.at[c], b1h_v)
    pltpu.sync_copy(w2h_hbm.at[c], w2h_v)
    pltpu.sync_copy(dm_hbm.at[pl.ds(base, SROW)], dm_v)

    def _epilogue(g_hbm):
        lanes = lax.iota(jnp.int32, 16)
        b1s = [b1h_v[pl.ds(j * 16, 16)] for j in range(DH // 16)]
        w2s = [w2h_v[pl.ds(j * 16, 16)] for j in range(DH // 16)]
        for t in range(SROW // SUB):
            off = base + t * SUB
            pltpu.sync_copy(acc_sh.at[pl.ds(off, SUB)], accb_v)
            pltpu.sync_copy(g_hbm.at[pl.ds(off, SUB)], gb_v)

            def gbody(rb, carry):
                r0 = rb * 16

                def rbody(i, qvec):
                    r = r0 + i
                    dvr = dm_v[t * SUB + r, pl.ds(0, 16)]
                    y = jnp.zeros((16,), jnp.float32)
                    for j in range(DH // 16):
                        sl = pl.ds(j * 16, 16)
                        z = (accb_v[r, sl] + gb_v[r, sl]) * dvr + b1s[j]
                        y = y + jnp.maximum(z, 0.0) * w2s[j]
                    qr = jnp.sum(y * dvr)
                    return jnp.where(lanes == i, qr, qvec)

                qvec = lax.fori_loop(0, 16, rbody, jnp.zeros((16,), jnp.float32))
                qp_v[pl.ds(t * SUB + r0, 16)] = qvec
                return carry

            lax.fori_loop(0, SUB // 16, gbody, 0)

    @pl.when(c == 0)
    def _ep0():
        _epilogue(g0_hbm)

    @pl.when(c == 1)
    def _ep1():
        _epilogue(g1_hbm)

    pltpu.sync_copy(qp_v, out_hbm.at[c, pl.ds(base, SROW)])


CH2 = 80
EPT = E // NW
NC2 = EPT // CH2


@functools.partial(
    pl.kernel,
    out_type=jax.ShapeDtypeStruct((NC, N), jnp.float32),
    mesh=_mesh,
    scratch_types=[
        pltpu.VMEM((EPT,), jnp.int32),
        pltpu.VMEM((NC2, CH2), jnp.int32),
        pltpu.VMEM((EPT,), jnp.float32),
        pltpu.VMEM((N,), jnp.float32),
        pltpu.VMEM((N,), jnp.float32),
        pltpu.VMEM_SHARED((N,), jnp.float32),
    ],
    compiler_params=pltpu.CompilerParams(
        use_tc_tiling_on_sc=False, needs_layout_passes=False
    ),
)
def _sagg_k(src_hbm, dst_hbm, q_hbm, out_hbm, src_v, dst_v, vals_v, q_v, zero_v, acc_sh):
    c = lax.axis_index("c")
    s = lax.axis_index("s")
    w = c * NS + s

    @pl.when(s == 0)
    def _zero():
        def zb(i, carry):
            zero_v[pl.ds(i * 16, 16)] = jnp.zeros((16,), jnp.float32)
            return carry

        lax.fori_loop(0, N // 16, zb, 0)
        pltpu.sync_copy(zero_v, acc_sh)

    plsc.subcore_barrier()
    pltpu.sync_copy(q_hbm.at[0], q_v)
    pltpu.sync_copy(q_hbm.at[1], zero_v)

    def qadd(i, carry):
        sl = pl.ds(i * 16, 16)
        q_v[sl] = q_v[sl] + zero_v[sl]
        return carry

    lax.fori_loop(0, N // 16, qadd, 0)
    pltpu.sync_copy(src_hbm.at[pl.ds(w * EPT, EPT)], src_v)
    pltpu.sync_copy(dst_hbm.at[w], dst_v)

    def gbody(i, carry):
        iv = src_v[pl.ds(i * 16, 16)]
        vals_v[pl.ds(i * 16, 16)] = plsc.load_gather(q_v, [iv])
        return carry

    lax.fori_loop(0, EPT // 16, gbody, 0)

    def sbody(j, carry):
        pltpu.sync_copy(vals_v.at[pl.ds(j * CH2, CH2)], acc_sh.at[dst_v.at[j]], add=True)
        return carry

    lax.fori_loop(0, NC2, sbody, 0)
    plsc.subcore_barrier()

    @pl.when(s == 0)
    def _out():
        pltpu.sync_copy(acc_sh, out_hbm.at[c])


def _mm_body(x_ref, w1_ref, h_ref):
    h_ref[...] = jnp.dot(x_ref[...], w1_ref[...], preferred_element_type=jnp.float32)


def _scale_body(h_ref, cnt_ref, g0_ref, g1_ref, dinv_ref, dm_ref):
    deg = cnt_ref[0, 0, 0, :] + cnt_ref[1, 0, 0, :] + 1.0
    dinv = lax.rsqrt(deg)
    g = h_ref[...] * dinv[:, None]
    g0_ref[...] = g[:, :DH]
    g1_ref[...] = g[:, DH:]
    dinv_ref[0, 0, :] = dinv
    dm_ref[...] = jnp.broadcast_to(dinv[:, None], (BLK, 16))


def _final_body(s_ref, q_ref, dinv_ref, b2_ref, out_ref):
    tot = s_ref[0, 0, 0, :] + s_ref[1, 0, 0, :] + q_ref[0, 0, 0, :] + q_ref[1, 0, 0, :]
    out_ref[0, 0, :] = jnp.tanh(dinv_ref[0, 0, :] * tot + b2_ref[0, 0])


_mm_call = pl.pallas_call(
    _mm_body,
    grid=(NB,),
    in_specs=[
        pl.BlockSpec((BLK, D), lambda i: (i, 0)),
        pl.BlockSpec((D, D), lambda i: (0, 0)),
    ],
    out_specs=pl.BlockSpec((BLK, D), lambda i: (i, 0)),
    out_shape=jax.ShapeDtypeStruct((N, D), jnp.float32),
)

_scale_call = pl.pallas_call(
    _scale_body,
    grid=(NB,),
    in_specs=[
        pl.BlockSpec((BLK, D), lambda i: (i, 0)),
        pl.BlockSpec((NC, 1, 1, BLK), lambda i: (0, i, 0, 0)),
    ],
    out_specs=[
        pl.BlockSpec((BLK, DH), lambda i: (i, 0)),
        pl.BlockSpec((BLK, DH), lambda i: (i, 0)),
        pl.BlockSpec((1, 1, BLK), lambda i: (i, 0, 0)),
        pl.BlockSpec((BLK, 16), lambda i: (i, 0)),
    ],
    out_shape=[
        jax.ShapeDtypeStruct((NPAD, DH), jnp.float32),
        jax.ShapeDtypeStruct((NPAD, DH), jnp.float32),
        jax.ShapeDtypeStruct((NB, 1, BLK), jnp.float32),
        jax.ShapeDtypeStruct((NPAD, 16), jnp.float32),
    ],
)

_final_call = pl.pallas_call(
    _final_body,
    grid=(NB,),
    in_specs=[
        pl.BlockSpec((NC, 1, 1, BLK), lambda i: (0, i, 0, 0)),
        pl.BlockSpec((NC, 1, 1, BLK), lambda i: (0, i, 0, 0)),
        pl.BlockSpec((1, 1, BLK), lambda i: (i, 0, 0)),
        pl.BlockSpec((1, 1), lambda i: (0, 0)),
    ],
    out_specs=pl.BlockSpec((1, 1, BLK), lambda i: (i, 0, 0)),
    out_shape=jax.ShapeDtypeStruct((NB, 1, BLK), jnp.float32),
)


def kernel(x, edge_index, W1, b1, W2, b2):
    ei = edge_index.astype(jnp.int32)
    edge3 = ei.reshape(2, ER, CH)

    counts = _count_k(edge3)
    h = _mm_call(x, W1)
    g0, g1, dinv3, dm = _scale_call(h, counts.reshape(NC, NB, 1, BLK))
    qp = _agg_k(edge3, g0, g1, dm,
                b1.reshape(NC, DH), W2.reshape(NC, DH))
    qs = qp[:, :N]
    s_part = _sagg_k(ei[0], ei[1].reshape(NW, NC2, CH2), qs)
    out3 = _final_call(
        s_part.reshape(NC, NB, 1, BLK),
        qs.reshape(NC, NB, 1, BLK),
        dinv3,
        b2.reshape(1, 1),
    )
    return out3.reshape(N, 1)


# speedup vs baseline: 1.3494x; 1.1292x over previous
import functools

import jax
import jax.numpy as jnp
from jax import lax
from jax.experimental import pallas as pl
from jax.experimental.pallas import tpu as pltpu, tpu_sc as plsc

N = 10000
E = 320000
D = 128
NC, NS = 2, 16
NW = NC * NS
CH = 125
ER = E // CH
NCH = ER // NW
NPAD = 10240
SROW = NPAD // NS

BLK = 1000
NB = N // BLK

_mesh = plsc.VectorSubcoreMesh(
    core_axis_name="c", subcore_axis_name="s", num_cores=NC, num_subcores=NS
)


@functools.partial(
    pl.kernel,
    out_type=jax.ShapeDtypeStruct((NC, N), jnp.float32),
    mesh=_mesh,
    scratch_types=[
        pltpu.VMEM((NCH, CH), jnp.int32),
        pltpu.VMEM((128,), jnp.float32),
        pltpu.VMEM((N,), jnp.float32),
        pltpu.VMEM_SHARED((N,), jnp.float32),
    ],
    compiler_params=pltpu.CompilerParams(use_tc_tiling_on_sc=False),
)
def _count_k(edge_hbm, out_hbm, idx_v, ones_v, zero_v, cnt_sh):
    c = lax.axis_index("c")
    s = lax.axis_index("s")
    w = c * NS + s
    for k in range(128 // 16):
        ones_v[pl.ds(k * 16, 16)] = jnp.ones((16,), jnp.float32)

    @pl.when(s == 0)
    def _zero():
        def zb(i, carry):
            zero_v[pl.ds(i * 16, 16)] = jnp.zeros((16,), jnp.float32)
            return carry

        lax.fori_loop(0, N // 16, zb, 0)
        pltpu.sync_copy(zero_v, cnt_sh)

    plsc.subcore_barrier()
    pltpu.sync_copy(edge_hbm.at[1, pl.ds(w * NCH, NCH)], idx_v)

    def body(j, carry):
        pltpu.sync_copy(ones_v.at[pl.ds(0, CH)], cnt_sh.at[idx_v.at[j]], add=True)
        return carry

    lax.fori_loop(0, NCH, body, 0)
    plsc.subcore_barrier()

    @pl.when(s == 0)
    def _out():
        pltpu.sync_copy(cnt_sh, out_hbm.at[c])


DH = D // 2
ECH = ER // NS


SUB = SROW // 8


@functools.partial(
    pl.kernel,
    out_type=jax.ShapeDtypeStruct((NC, NPAD), jnp.float32),
    mesh=_mesh,
    scratch_types=[
        pltpu.VMEM((ECH, CH), jnp.int32),
        pltpu.VMEM((ECH, CH), jnp.int32),
        pltpu.VMEM((CH, DH), jnp.float32),
        pltpu.VMEM((CH, DH), jnp.float32),
        pltpu.VMEM((CH, DH), jnp.float32),
        pltpu.VMEM((SUB, DH), jnp.float32),
        pltpu.VMEM((SUB, DH), jnp.float32),
        pltpu.VMEM((SROW, 16), jnp.float32),
        pltpu.VMEM((SROW,), jnp.float32),
        pltpu.VMEM((DH,), jnp.float32),
        pltpu.VMEM((DH,), jnp.float32),
        pltpu.VMEM_SHARED((NPAD, DH), jnp.float32),
        pltpu.SemaphoreType.DMA,
        pltpu.SemaphoreType.DMA,
        pltpu.SemaphoreType.DMA,
    ],
    compiler_params=pltpu.CompilerParams(
        use_tc_tiling_on_sc=False, needs_layout_passes=False
    ),
)
def _agg_k(edge_hbm, g0_hbm, g1_hbm, dm_hbm, b1h_hbm, w2h_hbm, out_hbm,
           src_v, dst_v, rows0_v, rows1_v, rows2_v, accb_v, gb_v, dm_v, qp_v,
           b1h_v, w2h_v, acc_sh, sem0, sem1, sem2):
    c = lax.axis_index("c")
    s = lax.axis_index("s")

    def zr(i, carry):
        def zk(k, carry2):
            accb_v[i, pl.ds(k * 16, 16)] = jnp.zeros((16,), jnp.float32)
            return carry2

        lax.fori_loop(0, DH // 16, zk, 0)
        return carry

    lax.fori_loop(0, SUB, zr, 0)
    for t in range(SROW // SUB):
        pltpu.sync_copy(accb_v, acc_sh.at[pl.ds(s * SROW + t * SUB, SUB)])
    plsc.subcore_barrier()

    pltpu.sync_copy(edge_hbm.at[0, pl.ds(s * ECH, ECH)], src_v)
    pltpu.sync_copy(edge_hbm.at[1, pl.ds(s * ECH, ECH)], dst_v)

    def _edge_loop(g_hbm):
        bufs = [(rows0_v, sem0), (rows1_v, sem1), (rows2_v, sem2)]
        for b, (rv, sm) in enumerate(bufs):
            pltpu.async_copy(g_hbm.at[src_v.at[b]], rv, sm)

        def step(j, rv, sm, issue_next):
            pltpu.make_async_copy(g_hbm.at[src_v.at[j]], rv, sm).wait()
            pltpu.sync_copy(rv, acc_sh.at[dst_v.at[j]], add=True)
            if issue_next:
                @pl.when(j + 3 < ECH)
                def _nx():
                    pltpu.async_copy(g_hbm.at[src_v.at[j + 3]], rv, sm)

        def body(jj, carry):
            j0 = 3 * jj
            for b, (rv, sm) in enumerate(bufs):
                step(j0 + b, rv, sm, True)
            return carry

        lax.fori_loop(0, ECH // 3, body, 0)
        for j in range(3 * (ECH // 3), ECH):
            step(j, *bufs[j % 3], False)

    @pl.when(c == 0)
    def _half0():
        _edge_loop(g0_hbm)

    @pl.when(c == 1)
    def _half1():
        _edge_loop(g1_hbm)

    plsc.subcore_barrier()

    base = s * SROW
    pltpu.sync_copy(b1h_hbm.at[c], b1h_v)
    pltpu.sync_copy(w2h_hbm.at[c], w2h_v)
    pltpu.sync_copy(dm_hbm.at[pl.ds(base, SROW)], dm_v)

    def _epilogue(g_hbm):
        lanes = lax.iota(jnp.int32, 16)
        b1s = [b1h_v[pl.ds(j * 16, 16)] for j in range(DH // 16)]
        w2s = [w2h_v[pl.ds(j * 16, 16)] for j in range(DH // 16)]
        for t in range(SROW // SUB):
            off = base + t * SUB
            pltpu.sync_copy(acc_sh.at[pl.ds(off, SUB)], accb_v)
            pltpu.sync_copy(g_hbm.at[pl.ds(off, SUB)], gb_v)

            def gbody(rb, carry):
                r0 = rb * 16

                def rbody(i, qvec):
                    r = r0 + i
                    dvr = dm_v[t * SUB + r, pl.ds(0, 16)]
                    y = jnp.zeros((16,), jnp.float32)
                    for j in range(DH // 16):
                        sl = pl.ds(j * 16, 16)
                        z = (accb_v[r, sl] + gb_v[r, sl]) * dvr + b1s[j]
                        y = y + jnp.maximum(z, 0.0) * w2s[j]
                    qr = jnp.sum(y * dvr)
                    return jnp.where(lanes == i, qr, qvec)

                qvec = lax.fori_loop(0, 16, rbody, jnp.zeros((16,), jnp.float32))
                qp_v[pl.ds(t * SUB + r0, 16)] = qvec
                return carry

            lax.fori_loop(0, SUB // 16, gbody, 0)

    @pl.when(c == 0)
    def _ep0():
        _epilogue(g0_hbm)

    @pl.when(c == 1)
    def _ep1():
        _epilogue(g1_hbm)

    pltpu.sync_copy(qp_v, out_hbm.at[c, pl.ds(base, SROW)])


CH2 = 80
EPT = E // NW
NC2 = EPT // CH2


@functools.partial(
    pl.kernel,
    out_type=jax.ShapeDtypeStruct((NC, N), jnp.float32),
    mesh=_mesh,
    scratch_types=[
        pltpu.VMEM((EPT,), jnp.int32),
        pltpu.VMEM((NC2, CH2), jnp.int32),
        pltpu.VMEM((EPT,), jnp.float32),
        pltpu.VMEM((N,), jnp.float32),
        pltpu.VMEM((N,), jnp.float32),
        pltpu.VMEM_SHARED((N,), jnp.float32),
    ],
    compiler_params=pltpu.CompilerParams(
        use_tc_tiling_on_sc=False, needs_layout_passes=False
    ),
)
def _sagg_k(src_hbm, dst_hbm, q_hbm, out_hbm, src_v, dst_v, vals_v, q_v, zero_v, acc_sh):
    c = lax.axis_index("c")
    s = lax.axis_index("s")
    w = c * NS + s

    @pl.when(s == 0)
    def _zero():
        def zb(i, carry):
            zero_v[pl.ds(i * 16, 16)] = jnp.zeros((16,), jnp.float32)
            return carry

        lax.fori_loop(0, N // 16, zb, 0)
        pltpu.sync_copy(zero_v, acc_sh)

    plsc.subcore_barrier()
    pltpu.sync_copy(q_hbm.at[0], q_v)
    pltpu.sync_copy(q_hbm.at[1], zero_v)

    def qadd(i, carry):
        sl = pl.ds(i * 16, 16)
        q_v[sl] = q_v[sl] + zero_v[sl]
        return carry

    lax.fori_loop(0, N // 16, qadd, 0)
    pltpu.sync_copy(src_hbm.at[pl.ds(w * EPT, EPT)], src_v)
    pltpu.sync_copy(dst_hbm.at[w], dst_v)

    def gbody(i, carry):
        iv = src_v[pl.ds(i * 16, 16)]
        vals_v[pl.ds(i * 16, 16)] = plsc.load_gather(q_v, [iv])
        return carry

    lax.fori_loop(0, EPT // 16, gbody, 0)

    def sbody(j, carry):
        pltpu.sync_copy(vals_v.at[pl.ds(j * CH2, CH2)], acc_sh.at[dst_v.at[j]], add=True)
        return carry

    lax.fori_loop(0, NC2, sbody, 0)
    plsc.subcore_barrier()

    @pl.when(s == 0)
    def _out():
        pltpu.sync_copy(acc_sh, out_hbm.at[c])


def _mm_body(x_ref, w1_ref, h_ref):
    h_ref[...] = jnp.dot(x_ref[...], w1_ref[...], preferred_element_type=jnp.float32)


def _scale_body(h_ref, cnt_ref, g0_ref, g1_ref, dinv_ref, dm_ref):
    deg = cnt_ref[0, 0, 0, :] + cnt_ref[1, 0, 0, :] + 1.0
    dinv = lax.rsqrt(deg)
    g = h_ref[...] * dinv[:, None]
    g0_ref[...] = g[:, :DH]
    g1_ref[...] = g[:, DH:]
    dinv_ref[0, 0, :] = dinv
    dm_ref[...] = jnp.broadcast_to(dinv[:, None], (BLK, 16))


def _final_body(s_ref, q_ref, dinv_ref, b2_ref, out_ref):
    tot = s_ref[0, 0, 0, :] + s_ref[1, 0, 0, :] + q_ref[0, 0, 0, :] + q_ref[1, 0, 0, :]
    out_ref[0, 0, :] = jnp.tanh(dinv_ref[0, 0, :] * tot + b2_ref[0, 0])


_mm_call = pl.pallas_call(
    _mm_body,
    grid=(NB,),
    in_specs=[
        pl.BlockSpec((BLK, D), lambda i: (i, 0)),
        pl.BlockSpec((D, D), lambda i: (0, 0)),
    ],
    out_specs=pl.BlockSpec((BLK, D), lambda i: (i, 0)),
    out_shape=jax.ShapeDtypeStruct((N, D), jnp.float32),
)

_scale_call = pl.pallas_call(
    _scale_body,
    grid=(NB,),
    in_specs=[
        pl.BlockSpec((BLK, D), lambda i: (i, 0)),
        pl.BlockSpec((NC, 1, 1, BLK), lambda i: (0, i, 0, 0)),
    ],
    out_specs=[
        pl.BlockSpec((BLK, DH), lambda i: (i, 0)),
        pl.BlockSpec((BLK, DH), lambda i: (i, 0)),
        pl.BlockSpec((1, 1, BLK), lambda i: (i, 0, 0)),
        pl.BlockSpec((BLK, 16), lambda i: (i, 0)),
    ],
    out_shape=[
        jax.ShapeDtypeStruct((NPAD, DH), jnp.float32),
        jax.ShapeDtypeStruct((NPAD, DH), jnp.float32),
        jax.ShapeDtypeStruct((NB, 1, BLK), jnp.float32),
        jax.ShapeDtypeStruct((NPAD, 16), jnp.float32),
    ],
)

_final_call = pl.pallas_call(
    _final_body,
    grid=(NB,),
    in_specs=[
        pl.BlockSpec((NC, 1, 1, BLK), lambda i: (0, i, 0, 0)),
        pl.BlockSpec((NC, 1, 1, BLK), lambda i: (0, i, 0, 0)),
        pl.BlockSpec((1, 1, BLK), lambda i: (i, 0, 0)),
        pl.BlockSpec((1, 1), lambda i: (0, 0)),
    ],
    out_specs=pl.BlockSpec((1, 1, BLK), lambda i: (i, 0, 0)),
    out_shape=jax.ShapeDtypeStruct((NB, 1, BLK), jnp.float32),
)


def kernel(x, edge_index, W1, b1, W2, b2):
    ei = edge_index.astype(jnp.int32)
    edge3 = ei.reshape(2, ER, CH)

    counts = _count_k(edge3)
    h = _mm_call(x, W1)
    g0, g1, dinv3, dm = _scale_call(h, counts.reshape(NC, NB, 1, BLK))
    qp = _agg_k(edge3, g0, g1, dm,
                b1.reshape(NC, DH), W2.reshape(NC, DH))
    qs = qp[:, :N]
    s_part = _sagg_k(ei[0], ei[1].reshape(NW, NC2, CH2), qs)
    out3 = _final_call(
        s_part.reshape(NC, NB, 1, BLK),
        qs.reshape(NC, NB, 1, BLK),
        dinv3,
        b2.reshape(1, 1),
    )
    return out3.reshape(N, 1)


# speedup vs baseline: 1.4413x; 1.0681x over previous
import functools

import jax
import jax.numpy as jnp
from jax import lax
from jax.experimental import pallas as pl
from jax.experimental.pallas import tpu as pltpu, tpu_sc as plsc

N = 10000
E = 320000
D = 128
NC, NS = 2, 16
NW = NC * NS
CH = 125
ER = E // CH
NCH = ER // NW
NPAD = 10240
SROW = NPAD // NS

BLK = 1000
NB = N // BLK

_mesh = plsc.VectorSubcoreMesh(
    core_axis_name="c", subcore_axis_name="s", num_cores=NC, num_subcores=NS
)


@functools.partial(
    pl.kernel,
    out_type=jax.ShapeDtypeStruct((NC, N), jnp.float32),
    mesh=_mesh,
    scratch_types=[
        pltpu.VMEM((NCH, CH), jnp.int32),
        pltpu.VMEM((128,), jnp.float32),
        pltpu.VMEM((N,), jnp.float32),
        pltpu.VMEM_SHARED((N,), jnp.float32),
        pltpu.SemaphoreType.DMA,
    ],
    compiler_params=pltpu.CompilerParams(use_tc_tiling_on_sc=False),
)
def _count_k(edge_hbm, out_hbm, idx_v, ones_v, zero_v, cnt_sh, semc):
    c = lax.axis_index("c")
    s = lax.axis_index("s")
    w = c * NS + s
    for k in range(128 // 16):
        ones_v[pl.ds(k * 16, 16)] = jnp.ones((16,), jnp.float32)

    @pl.when(s == 0)
    def _zero():
        def zb(i, carry):
            zero_v[pl.ds(i * 16, 16)] = jnp.zeros((16,), jnp.float32)
            return carry

        lax.fori_loop(0, N // 16, zb, 0)
        pltpu.sync_copy(zero_v, cnt_sh)

    plsc.subcore_barrier()
    pltpu.sync_copy(edge_hbm.at[1, pl.ds(w * NCH, NCH)], idx_v)

    def body(j, carry):
        pltpu.async_copy(ones_v.at[pl.ds(0, CH)], cnt_sh.at[idx_v.at[j]], semc, add=True)
        return carry

    lax.fori_loop(0, NCH, body, 0)

    def drain(j, carry):
        pltpu.make_async_copy(ones_v.at[pl.ds(0, CH)], cnt_sh.at[idx_v.at[j]], semc).wait()
        return carry

    lax.fori_loop(0, NCH, drain, 0)
    plsc.subcore_barrier()

    @pl.when(s == 0)
    def _out():
        pltpu.sync_copy(cnt_sh, out_hbm.at[c])


DH = D // 2
ECH = ER // NS


SUB = SROW // 8


@functools.partial(
    pl.kernel,
    out_type=jax.ShapeDtypeStruct((NC, NPAD), jnp.float32),
    mesh=_mesh,
    scratch_types=[
        pltpu.VMEM((ECH, CH), jnp.int32),
        pltpu.VMEM((ECH, CH), jnp.int32),
        pltpu.VMEM((CH, DH), jnp.float32),
        pltpu.VMEM((CH, DH), jnp.float32),
        pltpu.VMEM((CH, DH), jnp.float32),
        pltpu.VMEM((SUB, DH), jnp.float32),
        pltpu.VMEM((SUB, DH), jnp.float32),
        pltpu.VMEM((SROW, 16), jnp.float32),
        pltpu.VMEM((SROW,), jnp.float32),
        pltpu.VMEM((DH,), jnp.float32),
        pltpu.VMEM((DH,), jnp.float32),
        pltpu.VMEM_SHARED((NPAD, DH), jnp.float32),
        pltpu.SemaphoreType.DMA,
        pltpu.SemaphoreType.DMA,
        pltpu.SemaphoreType.DMA,
    ],
    compiler_params=pltpu.CompilerParams(
        use_tc_tiling_on_sc=False, needs_layout_passes=False
    ),
)
def _agg_k(edge_hbm, g0_hbm, g1_hbm, dm_hbm, b1h_hbm, w2h_hbm, out_hbm,
           src_v, dst_v, rows0_v, rows1_v, rows2_v, accb_v, gb_v, dm_v, qp_v,
           b1h_v, w2h_v, acc_sh, sem0, sem1, sem2):
    c = lax.axis_index("c")
    s = lax.axis_index("s")

    def zr(i, carry):
        def zk(k, carry2):
            accb_v[i, pl.ds(k * 16, 16)] = jnp.zeros((16,), jnp.float32)
            return carry2

        lax.fori_loop(0, DH // 16, zk, 0)
        return carry

    lax.fori_loop(0, SUB, zr, 0)
    for t in range(SROW // SUB):
        pltpu.sync_copy(accb_v, acc_sh.at[pl.ds(s * SROW + t * SUB, SUB)])
    plsc.subcore_barrier()

    pltpu.sync_copy(edge_hbm.at[0, pl.ds(s * ECH, ECH)], src_v)
    pltpu.sync_copy(edge_hbm.at[1, pl.ds(s * ECH, ECH)], dst_v)

    def _edge_loop(g_hbm):
        bufs = [(rows0_v, sem0), (rows1_v, sem1), (rows2_v, sem2)]
        for b, (rv, sm) in enumerate(bufs):
            pltpu.async_copy(g_hbm.at[src_v.at[b]], rv, sm)

        def step(j, rv, sm, issue_next):
            pltpu.make_async_copy(g_hbm.at[src_v.at[j]], rv, sm).wait()
            pltpu.sync_copy(rv, acc_sh.at[dst_v.at[j]], add=True)
            if issue_next:
                @pl.when(j + 3 < ECH)
                def _nx():
                    pltpu.async_copy(g_hbm.at[src_v.at[j + 3]], rv, sm)

        def body(jj, carry):
            j0 = 3 * jj
            for b, (rv, sm) in enumerate(bufs):
                step(j0 + b, rv, sm, True)
            return carry

        lax.fori_loop(0, ECH // 3, body, 0)
        for j in range(3 * (ECH // 3), ECH):
            step(j, *bufs[j % 3], False)

    @pl.when(c == 0)
    def _half0():
        _edge_loop(g0_hbm)

    @pl.when(c == 1)
    def _half1():
        _edge_loop(g1_hbm)

    plsc.subcore_barrier()

    base = s * SROW
    pltpu.sync_copy(b1h_hbm.at[c], b1h_v)
    pltpu.sync_copy(w2h_hbm.at[c], w2h_v)
    pltpu.sync_copy(dm_hbm.at[pl.ds(base, SROW)], dm_v)

    def _epilogue(g_hbm):
        lanes = lax.iota(jnp.int32, 16)
        b1s = [b1h_v[pl.ds(j * 16, 16)] for j in range(DH // 16)]
        w2s = [w2h_v[pl.ds(j * 16, 16)] for j in range(DH // 16)]
        for t in range(SROW // SUB):
            off = base + t * SUB
            pltpu.sync_copy(acc_sh.at[pl.ds(off, SUB)], accb_v)
            pltpu.sync_copy(g_hbm.at[pl.ds(off, SUB)], gb_v)

            def gbody(rb, carry):
                r0 = rb * 16

                def rbody(i, qvec):
                    r = r0 + i
                    dvr = dm_v[t * SUB + r, pl.ds(0, 16)]
                    y = jnp.zeros((16,), jnp.float32)
                    for j in range(DH // 16):
                        sl = pl.ds(j * 16, 16)
                        z = (accb_v[r, sl] + gb_v[r, sl]) * dvr + b1s[j]
                        y = y + jnp.maximum(z, 0.0) * w2s[j]
                    qr = jnp.sum(y * dvr)
                    return jnp.where(lanes == i, qr, qvec)

                qvec = lax.fori_loop(0, 16, rbody, jnp.zeros((16,), jnp.float32))
                qp_v[pl.ds(t * SUB + r0, 16)] = qvec
                return carry

            lax.fori_loop(0, SUB // 16, gbody, 0)

    @pl.when(c == 0)
    def _ep0():
        _epilogue(g0_hbm)

    @pl.when(c == 1)
    def _ep1():
        _epilogue(g1_hbm)

    pltpu.sync_copy(qp_v, out_hbm.at[c, pl.ds(base, SROW)])


CH2 = 80
EPT = E // NW
NC2 = EPT // CH2


@functools.partial(
    pl.kernel,
    out_type=jax.ShapeDtypeStruct((NC, N), jnp.float32),
    mesh=_mesh,
    scratch_types=[
        pltpu.VMEM((EPT,), jnp.int32),
        pltpu.VMEM((NC2, CH2), jnp.int32),
        pltpu.VMEM((EPT,), jnp.float32),
        pltpu.VMEM((N,), jnp.float32),
        pltpu.VMEM((N,), jnp.float32),
        pltpu.VMEM_SHARED((N,), jnp.float32),
        pltpu.SemaphoreType.DMA,
    ],
    compiler_params=pltpu.CompilerParams(
        use_tc_tiling_on_sc=False, needs_layout_passes=False
    ),
)
def _sagg_k(src_hbm, dst_hbm, q_hbm, out_hbm, src_v, dst_v, vals_v, q_v, zero_v, acc_sh, semq):
    c = lax.axis_index("c")
    s = lax.axis_index("s")
    w = c * NS + s

    @pl.when(s == 0)
    def _zero():
        def zb(i, carry):
            zero_v[pl.ds(i * 16, 16)] = jnp.zeros((16,), jnp.float32)
            return carry

        lax.fori_loop(0, N // 16, zb, 0)
        pltpu.sync_copy(zero_v, acc_sh)

    plsc.subcore_barrier()
    pltpu.sync_copy(q_hbm.at[0], q_v)
    pltpu.sync_copy(q_hbm.at[1], zero_v)

    def qadd(i, carry):
        sl = pl.ds(i * 16, 16)
        q_v[sl] = q_v[sl] + zero_v[sl]
        return carry

    lax.fori_loop(0, N // 16, qadd, 0)
    pltpu.sync_copy(src_hbm.at[pl.ds(w * EPT, EPT)], src_v)
    pltpu.sync_copy(dst_hbm.at[w], dst_v)

    def gbody(i, carry):
        iv = src_v[pl.ds(i * 16, 16)]
        vals_v[pl.ds(i * 16, 16)] = plsc.load_gather(q_v, [iv])
        return carry

    lax.fori_loop(0, EPT // 16, gbody, 0)

    def sbody(j, carry):
        pltpu.async_copy(vals_v.at[pl.ds(j * CH2, CH2)], acc_sh.at[dst_v.at[j]], semq, add=True)
        return carry

    lax.fori_loop(0, NC2, sbody, 0)

    def sdrain(j, carry):
        pltpu.make_async_copy(vals_v.at[pl.ds(j * CH2, CH2)], acc_sh.at[dst_v.at[j]], semq).wait()
        return carry

    lax.fori_loop(0, NC2, sdrain, 0)
    plsc.subcore_barrier()

    @pl.when(s == 0)
    def _out():
        pltpu.sync_copy(acc_sh, out_hbm.at[c])


def _mm_body(x_ref, w1_ref, h_ref):
    h_ref[...] = jnp.dot(x_ref[...], w1_ref[...], preferred_element_type=jnp.float32)


def _scale_body(h_ref, cnt_ref, g0_ref, g1_ref, dinv_ref, dm_ref):
    deg = cnt_ref[0, 0, 0, :] + cnt_ref[1, 0, 0, :] + 1.0
    dinv = lax.rsqrt(deg)
    g = h_ref[...] * dinv[:, None]
    g0_ref[...] = g[:, :DH]
    g1_ref[...] = g[:, DH:]
    dinv_ref[0, 0, :] = dinv
    dm_ref[...] = jnp.broadcast_to(dinv[:, None], (BLK, 16))


def _final_body(s_ref, q_ref, dinv_ref, b2_ref, out_ref):
    tot = s_ref[0, 0, 0, :] + s_ref[1, 0, 0, :] + q_ref[0, 0, 0, :] + q_ref[1, 0, 0, :]
    out_ref[0, 0, :] = jnp.tanh(dinv_ref[0, 0, :] * tot + b2_ref[0, 0])


_mm_call = pl.pallas_call(
    _mm_body,
    grid=(NB,),
    in_specs=[
        pl.BlockSpec((BLK, D), lambda i: (i, 0)),
        pl.BlockSpec((D, D), lambda i: (0, 0)),
    ],
    out_specs=pl.BlockSpec((BLK, D), lambda i: (i, 0)),
    out_shape=jax.ShapeDtypeStruct((N, D), jnp.float32),
)

_scale_call = pl.pallas_call(
    _scale_body,
    grid=(NB,),
    in_specs=[
        pl.BlockSpec((BLK, D), lambda i: (i, 0)),
        pl.BlockSpec((NC, 1, 1, BLK), lambda i: (0, i, 0, 0)),
    ],
    out_specs=[
        pl.BlockSpec((BLK, DH), lambda i: (i, 0)),
        pl.BlockSpec((BLK, DH), lambda i: (i, 0)),
        pl.BlockSpec((1, 1, BLK), lambda i: (i, 0, 0)),
        pl.BlockSpec((BLK, 16), lambda i: (i, 0)),
    ],
    out_shape=[
        jax.ShapeDtypeStruct((NPAD, DH), jnp.float32),
        jax.ShapeDtypeStruct((NPAD, DH), jnp.float32),
        jax.ShapeDtypeStruct((NB, 1, BLK), jnp.float32),
        jax.ShapeDtypeStruct((NPAD, 16), jnp.float32),
    ],
)

_final_call = pl.pallas_call(
    _final_body,
    grid=(NB,),
    in_specs=[
        pl.BlockSpec((NC, 1, 1, BLK), lambda i: (0, i, 0, 0)),
        pl.BlockSpec((NC, 1, 1, BLK), lambda i: (0, i, 0, 0)),
        pl.BlockSpec((1, 1, BLK), lambda i: (i, 0, 0)),
        pl.BlockSpec((1, 1), lambda i: (0, 0)),
    ],
    out_specs=pl.BlockSpec((1, 1, BLK), lambda i: (i, 0, 0)),
    out_shape=jax.ShapeDtypeStruct((NB, 1, BLK), jnp.float32),
)


def kernel(x, edge_index, W1, b1, W2, b2):
    ei = edge_index.astype(jnp.int32)
    edge3 = ei.reshape(2, ER, CH)

    counts = _count_k(edge3)
    h = _mm_call(x, W1)
    g0, g1, dinv3, dm = _scale_call(h, counts.reshape(NC, NB, 1, BLK))
    qp = _agg_k(edge3, g0, g1, dm,
                b1.reshape(NC, DH), W2.reshape(NC, DH))
    qs = qp[:, :N]
    s_part = _sagg_k(ei[0], ei[1].reshape(NW, NC2, CH2), qs)
    out3 = _final_call(
        s_part.reshape(NC, NB, 1, BLK),
        qs.reshape(NC, NB, 1, BLK),
        dinv3,
        b2.reshape(1, 1),
    )
    return out3.reshape(N, 1)
